# baseline (speedup 1.0000x reference)
import jax
import jax.numpy as jnp
from jax import lax
from jax.experimental import pallas as pl
from jax.experimental.pallas import tpu as pltpu
from jax.experimental.pallas import tpu_sc as plsc

B = 16384
D = 64
NROWS = 1000000
NC = 2
NS = 16
NW = NC * NS
BPW = B // NW
RND = 2
BPR = BPW // RND
OPW = BPW * D // 128
OPR = OPW // RND
CHUNK = 128
LANES = 16


def _gmf_body(user_ids, movie_ids, user_table, item_table, out,
              idx_u, idx_m, q_u, q_m, gat_u, gat_m, out_v, sem_u, sem_m):
    wid = lax.axis_index("s") * NC + lax.axis_index("c")
    base = wid * BPW

    pltpu.sync_copy(user_ids.at[pl.ds(base, BPW)], idx_u)
    pltpu.sync_copy(movie_ids.at[pl.ds(base, BPW)], idx_m)

    def shift(ch, carry):
        sl = pl.ds(ch * LANES, LANES)
        q_u[sl] = jax.lax.shift_right_logical(idx_u[sl], 1)
        q_m[sl] = jax.lax.shift_right_logical(idx_m[sl], 1)
        return carry

    lax.fori_loop(0, BPW // LANES, shift, 0)

    for r in range(RND):
        waits = []
        for j in range(BPR // CHUNK):
            o = r * BPR + j * CHUNK
            waits.append(pltpu.async_copy(
                user_table.at[q_u.at[pl.ds(o, CHUNK)]],
                gat_u.at[pl.ds(j * CHUNK, CHUNK)], sem_u))
            waits.append(pltpu.async_copy(
                item_table.at[q_m.at[pl.ds(o, CHUNK)]],
                gat_m.at[pl.ds(j * CHUNK, CHUNK)], sem_m))
        for w in waits:
            w.wait()

        def mul16(g, carry):
            i0 = g * LANES
            vu = idx_u[pl.ds(r * BPR + i0, LANES)]
            vm = idx_m[pl.ds(r * BPR + i0, LANES)]
            for j in range(LANES):
                i = i0 + j
                hu = (vu[j] & 1) * D
                hm = (vm[j] & 1) * D
                orow = r * OPR + i // 2
                for c in range(D // LANES):
                    osl = pl.ds((i % 2) * D + c * LANES, LANES)
                    out_v[orow, osl] = (gat_u[i, pl.ds(hu + c * LANES, LANES)]
                                        * gat_m[i, pl.ds(hm + c * LANES, LANES)])
            return carry

        lax.fori_loop(0, BPR // LANES, mul16, 0)

    pltpu.sync_copy(out_v, out.at[pl.ds(wid * OPW, OPW)])


def kernel(user_ids, movie_ids, user_table, item_table):
    uw = user_table[:NROWS].reshape(NROWS // 2, 128)
    mw = item_table[:NROWS].reshape(NROWS // 2, 128)
    mesh = plsc.VectorSubcoreMesh(core_axis_name="c", subcore_axis_name="s")
    run = pl.kernel(
        _gmf_body,
        mesh=mesh,
        compiler_params=pltpu.CompilerParams(use_tc_tiling_on_sc=True),
        out_type=jax.ShapeDtypeStruct((B * D // 128, 128), jnp.float32),
        scratch_types=[
            pltpu.VMEM((BPW,), jnp.int32),
            pltpu.VMEM((BPW,), jnp.int32),
            pltpu.VMEM((BPW,), jnp.int32),
            pltpu.VMEM((BPW,), jnp.int32),
            pltpu.VMEM((BPR, 128), jnp.float32),
            pltpu.VMEM((BPR, 128), jnp.float32),
            pltpu.VMEM((OPW, 128), jnp.float32),
            pltpu.SemaphoreType.DMA,
            pltpu.SemaphoreType.DMA,
        ],
    )
    flat = run(user_ids.astype(jnp.int32), movie_ids.astype(jnp.int32), uw, mw)
    return flat.reshape(B, D)
